# baseline (speedup 1.0000x reference)
import jax
import jax.numpy as jnp
from jax import lax
from jax.experimental import pallas as pl
from jax.experimental.pallas import tpu as pltpu
from jax.experimental.pallas import tpu_sc as plsc

T = 8192
H = 1024
K = 2
E = 64
S = T * K
NC = 2
NS = 16
NW = NC * NS
L = 16
CH = S // NW
NV = CH // L
TPW = T // NW
CC = 8
NCH = TPW // CC
NBUF = 6

_mesh = dict(core_axis_name="c", subcore_axis_name="s", num_cores=NC,
             num_subcores=NS)


def _slot_vec(e_v, row0, i, lane):
    sloc = i * L + lane
    return plsc.load_gather(e_v, [row0 + (sloc >> 1), sloc & 1])


def _moe_kernel(ti_hbm, x_hbm, out_hbm, splits_hbm, hist_hbm,
                e_v, hist_v, all_hist_v, tot_v, off_v, dest_v,
                rows0, rows1, rows2, rows3, rows4, rows5,
                sem_e0, sem_o0, sem_e1, sem_o1, sem_e2, sem_o2,
                sem_e3, sem_o3, sem_e4, sem_o4, sem_e5, sem_o5,
                sem_l0, sem_l1, sem_l2, sem_l3, sem_l4, sem_l5):
    c = lax.axis_index("c")
    s = lax.axis_index("s")
    g = c * NS + s
    lane = lax.iota(jnp.int32, L)

    bufs = (rows0, rows1, rows2, rows3, rows4, rows5)
    lsems = (sem_l0, sem_l1, sem_l2, sem_l3, sem_l4, sem_l5)

    def _load(ch):
        q = ch % NBUF
        return pltpu.async_copy(
            x_hbm.at[pl.ds(g * TPW + ch * CC, CC)], bufs[q], lsems[q])

    lh = [None] * NBUF
    for ch in range(NBUF - 1):
        lh[ch] = _load(ch)

    pltpu.sync_copy(ti_hbm.at[pl.ds(s * (CH // K), CH // K)],
                    e_v.at[pl.ds(0, CH // K)])
    pltpu.sync_copy(ti_hbm.at[pl.ds((s + NS) * (CH // K), CH // K)],
                    e_v.at[pl.ds(CH // K, CH // K)])
    zeros = jnp.zeros((L,), jnp.int32)
    for half in range(2):
        for j in range(E // L):
            hist_v[pl.ds(L * j, L)] = zeros

        def hist_body(i, _, half=half):
            ev = _slot_vec(e_v, half * (CH // K), i, lane)
            cnt, last = plsc.scan_count(ev)
            plsc.addupdate_scatter(hist_v, [ev], cnt, mask=last)
            return 0

        lax.fori_loop(0, NV, hist_body, 0)
        pltpu.sync_copy(hist_v, hist_hbm.at[c, s + half * NS])

    plsc.subcore_barrier()
    pltpu.sync_copy(hist_hbm.at[c], all_hist_v)

    carry = jnp.int32(0)
    for j in range(E // L):
        sl = pl.ds(L * j, L)

        def acc_body(gp, tm, sl=sl):
            tot, mine = tm
            h = all_hist_v[gp, sl]
            return tot + h, mine + h * (gp < g).astype(jnp.int32)

        tot, mine = lax.fori_loop(
            0, NW, acc_body,
            (jnp.zeros((L,), jnp.int32), jnp.zeros((L,), jnp.int32)))
        tot_v[sl] = tot
        csum = plsc.cumsum(tot)
        off_v[sl] = mine + (csum - tot) + carry
        carry = carry + jnp.sum(tot)

    @pl.when(g == 0)
    def _():
        pltpu.sync_copy(tot_v, splits_hbm)

    def rank_body(i, _):
        ev = _slot_vec(e_v, c * (CH // K), i, lane)
        cnt, last = plsc.scan_count(ev)
        base = plsc.load_gather(off_v, [ev])
        dest = base + cnt - 1
        plsc.addupdate_scatter(off_v, [ev], cnt, mask=last)
        sloc = i * L + lane
        t_loc = sloc >> 1
        row = (sloc & 1) * NCH + (t_loc // CC)
        plsc.store_scatter(dest_v, [row, t_loc & (CC - 1)], dest)
        return 0

    lax.fori_loop(0, NV, rank_body, 0)

    ssems = ((sem_e0, sem_o0), (sem_e1, sem_o1), (sem_e2, sem_o2),
             (sem_e3, sem_o3), (sem_e4, sem_o4), (sem_e5, sem_o5))
    sh = [None] * NBUF
    for ch in range(NCH):
        p = ch % NBUF
        lh[p].wait()
        h_e = pltpu.async_copy(bufs[p], out_hbm.at[dest_v.at[ch]],
                               ssems[p][0])
        h_o = pltpu.async_copy(bufs[p], out_hbm.at[dest_v.at[NCH + ch]],
                               ssems[p][1])
        nxt = ch + NBUF - 1
        if nxt < NCH:
            q = nxt % NBUF
            if sh[q] is not None:
                sh[q][0].wait()
                sh[q][1].wait()
            lh[q] = _load(nxt)
        sh[p] = (h_e, h_o)
    for p in range(NBUF):
        if sh[p] is not None:
            sh[p][0].wait()
            sh[p][1].wait()


def _combine_body(x_ref, w_ref, o_ref):
    wsum = jnp.sum(w_ref[...], axis=1, keepdims=True)
    o_ref[...] = x_ref[...] * wsum


def kernel(x, topk_indices, topk_weights):
    moe = pl.kernel(
        _moe_kernel,
        out_type=(jax.ShapeDtypeStruct((S, H), jnp.float32),
                  jax.ShapeDtypeStruct((E,), jnp.int32),
                  jax.ShapeDtypeStruct((NC, NW, E), jnp.int32)),
        mesh=plsc.VectorSubcoreMesh(**_mesh),
        compiler_params=pltpu.CompilerParams(needs_layout_passes=False),
        scratch_types=[
            pltpu.VMEM((CH, K), jnp.int32),
            pltpu.VMEM((E,), jnp.int32),
            pltpu.VMEM((NW, E), jnp.int32),
            pltpu.VMEM((E,), jnp.int32),
            pltpu.VMEM((E,), jnp.int32),
            pltpu.VMEM((2 * NCH, CC), jnp.int32),
            pltpu.VMEM((CC, H), jnp.float32),
            pltpu.VMEM((CC, H), jnp.float32),
            pltpu.VMEM((CC, H), jnp.float32),
            pltpu.VMEM((CC, H), jnp.float32),
            pltpu.VMEM((CC, H), jnp.float32),
            pltpu.VMEM((CC, H), jnp.float32),
        ] + [pltpu.SemaphoreType.DMA] * 18,
    )
    dispatched, splits, _ = moe(topk_indices, x)

    combined = pl.pallas_call(
        _combine_body,
        grid=(T // 512,),
        in_specs=[pl.BlockSpec((512, H), lambda i: (i, 0)),
                  pl.BlockSpec((512, K), lambda i: (i, 0))],
        out_specs=pl.BlockSpec((512, H), lambda i: (i, 0)),
        out_shape=jax.ShapeDtypeStruct((T, H), jnp.float32),
    )(x, topk_weights)

    return combined, dispatched, splits


# speedup vs baseline: 1.0041x; 1.0041x over previous
import jax
import jax.numpy as jnp
from jax import lax
from jax.experimental import pallas as pl
from jax.experimental.pallas import tpu as pltpu
from jax.experimental.pallas import tpu_sc as plsc

T = 8192
H = 1024
K = 2
E = 64
S = T * K
NC = 2
NS = 16
NW = NC * NS
L = 16
CH = S // NW
NV = CH // L
TPW = T // NW
CC = 16
NCH = TPW // CC
NBUF = 3

_mesh = dict(core_axis_name="c", subcore_axis_name="s", num_cores=NC,
             num_subcores=NS)


def _slot_vec(e_v, row0, i, lane):
    sloc = i * L + lane
    return plsc.load_gather(e_v, [row0 + (sloc >> 1), sloc & 1])


def _moe_kernel(ti_hbm, x_hbm, out_hbm, splits_hbm, hist_hbm,
                e_v, hist0_v, hist1_v, all_hist_v, tot_v, off_v, dest_v,
                rows0, rows1, rows2,
                sem_e0, sem_o0, sem_e1, sem_o1, sem_e2, sem_o2,
                sem_l0, sem_l1, sem_l2, sem_t0, sem_t1, sem_h):
    c = lax.axis_index("c")
    s = lax.axis_index("s")
    g = c * NS + s
    lane = lax.iota(jnp.int32, L)

    bufs = (rows0, rows1, rows2)
    lsems = (sem_l0, sem_l1, sem_l2)

    def _load(ch):
        q = ch % NBUF
        return pltpu.async_copy(
            x_hbm.at[pl.ds(g * TPW + ch * CC, CC)], bufs[q], lsems[q])

    lh = [None] * NBUF
    for ch in range(NBUF - 1):
        lh[ch] = _load(ch)

    th0 = pltpu.async_copy(ti_hbm.at[pl.ds(s * (CH // K), CH // K)],
                           e_v.at[pl.ds(0, CH // K)], sem_t0)
    th1 = pltpu.async_copy(ti_hbm.at[pl.ds((s + NS) * (CH // K), CH // K)],
                           e_v.at[pl.ds(CH // K, CH // K)], sem_t1)
    th0.wait()
    th1.wait()
    zeros = jnp.zeros((L,), jnp.int32)
    for half in range(2):
        hist_v = hist0_v if half == 0 else hist1_v
        for j in range(E // L):
            hist_v[pl.ds(L * j, L)] = zeros

        def hist_body(i, _, half=half, hist_v=hist_v):
            ev = _slot_vec(e_v, half * (CH // K), i, lane)
            cnt, last = plsc.scan_count(ev)
            plsc.addupdate_scatter(hist_v, [ev], cnt, mask=last)
            return 0

        lax.fori_loop(0, NV, hist_body, 0)
        if half == 0:
            hh0 = pltpu.async_copy(hist0_v, hist_hbm.at[c, s], sem_h)
        else:
            hh1 = pltpu.async_copy(hist1_v, hist_hbm.at[c, s + NS], sem_h)

    hh0.wait()
    hh1.wait()
    plsc.subcore_barrier()
    pltpu.sync_copy(hist_hbm.at[c], all_hist_v)

    carry = jnp.int32(0)
    for j in range(E // L):
        sl = pl.ds(L * j, L)

        def acc_body(gp, tm, sl=sl):
            tot, mine = tm
            h = all_hist_v[gp, sl]
            return tot + h, mine + h * (gp < g).astype(jnp.int32)

        tot, mine = lax.fori_loop(
            0, NW, acc_body,
            (jnp.zeros((L,), jnp.int32), jnp.zeros((L,), jnp.int32)))
        tot_v[sl] = tot
        csum = plsc.cumsum(tot)
        off_v[sl] = mine + (csum - tot) + carry
        carry = carry + jnp.sum(tot)

    def rank_body(i, _):
        ev = _slot_vec(e_v, c * (CH // K), i, lane)
        cnt, last = plsc.scan_count(ev)
        base = plsc.load_gather(off_v, [ev])
        dest = base + cnt - 1
        plsc.addupdate_scatter(off_v, [ev], cnt, mask=last)
        sloc = i * L + lane
        t_loc = sloc >> 1
        row = (sloc & 1) * NCH + (t_loc // CC)
        plsc.store_scatter(dest_v, [row, t_loc & (CC - 1)], dest)
        return 0

    lax.fori_loop(0, NV, rank_body, 0)

    ssems = ((sem_e0, sem_o0), (sem_e1, sem_o1), (sem_e2, sem_o2))
    sh = [None] * NBUF
    for ch in range(NCH):
        p = ch % NBUF
        lh[p].wait()
        h_e = pltpu.async_copy(bufs[p], out_hbm.at[dest_v.at[ch]],
                               ssems[p][0])
        h_o = pltpu.async_copy(bufs[p], out_hbm.at[dest_v.at[NCH + ch]],
                               ssems[p][1])
        nxt = ch + NBUF - 1
        if nxt < NCH:
            q = nxt % NBUF
            if sh[q] is not None:
                sh[q][0].wait()
                sh[q][1].wait()
            lh[q] = _load(nxt)
        sh[p] = (h_e, h_o)
    @pl.when(g == 0)
    def _():
        pltpu.sync_copy(tot_v, splits_hbm)

    for p in range(NBUF):
        if sh[p] is not None:
            sh[p][0].wait()
            sh[p][1].wait()


def _combine_body(x_ref, w_ref, o_ref):
    wsum = jnp.sum(w_ref[...], axis=1, keepdims=True)
    o_ref[...] = x_ref[...] * wsum


def kernel(x, topk_indices, topk_weights):
    moe = pl.kernel(
        _moe_kernel,
        out_type=(jax.ShapeDtypeStruct((S, H), jnp.float32),
                  jax.ShapeDtypeStruct((E,), jnp.int32),
                  jax.ShapeDtypeStruct((NC, NW, E), jnp.int32)),
        mesh=plsc.VectorSubcoreMesh(**_mesh),
        compiler_params=pltpu.CompilerParams(needs_layout_passes=False),
        scratch_types=[
            pltpu.VMEM((CH, K), jnp.int32),
            pltpu.VMEM((E,), jnp.int32),
            pltpu.VMEM((E,), jnp.int32),
            pltpu.VMEM((NW, E), jnp.int32),
            pltpu.VMEM((E,), jnp.int32),
            pltpu.VMEM((E,), jnp.int32),
            pltpu.VMEM((2 * NCH, CC), jnp.int32),
            pltpu.VMEM((CC, H), jnp.float32),
            pltpu.VMEM((CC, H), jnp.float32),
            pltpu.VMEM((CC, H), jnp.float32),
        ] + [pltpu.SemaphoreType.DMA] * 12,
    )
    dispatched, splits, _ = moe(topk_indices, x)

    combined = pl.pallas_call(
        _combine_body,
        grid=(T // 512,),
        in_specs=[pl.BlockSpec((512, H), lambda i: (i, 0)),
                  pl.BlockSpec((512, K), lambda i: (i, 0))],
        out_specs=pl.BlockSpec((512, H), lambda i: (i, 0)),
        out_shape=jax.ShapeDtypeStruct((T, H), jnp.float32),
    )(x, topk_weights)

    return combined, dispatched, splits
